# Initial kernel scaffold; baseline (speedup 1.0000x reference)
import jax
import jax.numpy as jnp
from jax import lax
from jax.experimental import pallas as pl
from jax.experimental.pallas import tpu as pltpu
from jax.experimental.pallas import tpu_sc as plsc

NUM_ROWS = 100000
DIM = 64

NC = 2
NS = 16
NW = NC * NS

CHUNK = 128


def _make_kernel(total):
    per_w = total // NW
    n_chunks = per_w // CHUNK

    mesh = plsc.VectorSubcoreMesh(core_axis_name="c", subcore_axis_name="s")

    @pl.kernel(
        out_type=jax.ShapeDtypeStruct((total, DIM), jnp.float32),
        mesh=mesh,
        scratch_types=[
            pltpu.VMEM((n_chunks, CHUNK), jnp.int32),
            pltpu.VMEM((CHUNK, DIM), jnp.float32),
            pltpu.SemaphoreType.DMA,
        ],
    )
    def k(table_hbm, idx_hbm, out_hbm, idx_v, buf, gsem):
        wid = lax.axis_index("s") * NC + lax.axis_index("c")
        base_chunk = wid * n_chunks
        pltpu.sync_copy(idx_hbm.at[pl.ds(base_chunk, n_chunks)], idx_v)

        @pl.loop(0, n_chunks)
        def _(j):
            pltpu.async_copy(table_hbm.at[idx_v.at[j]], buf, gsem).wait()
            pltpu.sync_copy(
                buf, out_hbm.at[pl.ds((base_chunk + j) * CHUNK, CHUNK)]
            )

    return k


def kernel(inputs, table):
    b, s = inputs.shape
    total = b * s
    idx2d = inputs.reshape(total // CHUNK, CHUNK).astype(jnp.int32)
    out = _make_kernel(total)(table, idx2d)
    return out.reshape(b, s, DIM)


# speedup vs baseline: 3.5494x; 3.5494x over previous
import jax
import jax.numpy as jnp
from jax import lax
from jax.experimental import pallas as pl
from jax.experimental.pallas import tpu as pltpu
from jax.experimental.pallas import tpu_sc as plsc

NUM_ROWS = 100000
DIM = 64

NC = 2
NS = 16
NW = NC * NS

CHUNK = 128


def _make_kernel(total):
    per_w = total // NW
    n_chunks = per_w // CHUNK

    mesh = plsc.VectorSubcoreMesh(core_axis_name="c", subcore_axis_name="s")

    @pl.kernel(
        out_type=jax.ShapeDtypeStruct((total, DIM), jnp.float32),
        mesh=mesh,
        scratch_types=[
            pltpu.VMEM((n_chunks, CHUNK), jnp.int32),
            pltpu.VMEM((CHUNK, DIM), jnp.float32),
            pltpu.SemaphoreType.DMA,
        ],
        compiler_params=pltpu.CompilerParams(use_tc_tiling_on_sc=False),
    )
    def k(table_hbm, idx_hbm, out_hbm, idx_v, buf, gsem):
        wid = lax.axis_index("s") * NC + lax.axis_index("c")
        base_chunk = wid * n_chunks
        pltpu.sync_copy(idx_hbm.at[pl.ds(base_chunk, n_chunks)], idx_v)

        @pl.loop(0, n_chunks)
        def _(j):
            pltpu.async_copy(table_hbm.at[idx_v.at[j]], buf, gsem).wait()
            pltpu.sync_copy(
                buf, out_hbm.at[pl.ds((base_chunk + j) * CHUNK, CHUNK)]
            )

    return k


def kernel(inputs, table):
    b, s = inputs.shape
    total = b * s
    idx2d = inputs.reshape(total // CHUNK, CHUNK).astype(jnp.int32)
    out = _make_kernel(total)(table, idx2d)
    return out.reshape(b, s, DIM)


# speedup vs baseline: 4.2666x; 1.2021x over previous
import jax
import jax.numpy as jnp
from jax import lax
from jax.experimental import pallas as pl
from jax.experimental.pallas import tpu as pltpu
from jax.experimental.pallas import tpu_sc as plsc

NUM_ROWS = 100000
DIM = 64

NC = 2
NS = 16
NW = NC * NS

CHUNK = 128

RING = 8


def _make_kernel(total):
    per_w = total // NW
    n_chunks = per_w // CHUNK
    assert (n_chunks - RING) % RING == 0

    mesh = plsc.VectorSubcoreMesh(core_axis_name="c", subcore_axis_name="s")

    @pl.kernel(
        out_type=jax.ShapeDtypeStruct((total, DIM), jnp.float32),
        mesh=mesh,
        scratch_types=[
            pltpu.VMEM((n_chunks, CHUNK), jnp.int32),
            pltpu.VMEM((RING, CHUNK, DIM), jnp.float32),
        ]
        + [pltpu.SemaphoreType.DMA] * (2 * RING),
        compiler_params=pltpu.CompilerParams(use_tc_tiling_on_sc=False),
    )
    def k(table_hbm, idx_hbm, out_hbm, idx_v, bufs, *sems):
        gsem = sems[:RING]
        osem = sems[RING:]
        wid = lax.axis_index("s") * NC + lax.axis_index("c")
        base_chunk = wid * n_chunks
        pltpu.sync_copy(idx_hbm.at[pl.ds(base_chunk, n_chunks)], idx_v)

        def fire_gather(j, slot):
            pltpu.async_copy(table_hbm.at[idx_v.at[j]], bufs.at[slot],
                             gsem[slot])

        def wait_gather(slot):
            pltpu.make_async_copy(
                table_hbm.at[idx_v.at[0]], bufs.at[slot], gsem[slot]
            ).wait()

        def fire_out(j, slot):
            pltpu.async_copy(
                bufs.at[slot],
                out_hbm.at[pl.ds((base_chunk + j) * CHUNK, CHUNK)],
                osem[slot],
            )

        def wait_out(slot):
            pltpu.make_async_copy(
                bufs.at[slot],
                out_hbm.at[pl.ds(base_chunk * CHUNK, CHUNK)],
                osem[slot],
            ).wait()

        for t in range(RING - 1):
            fire_gather(t, t)
        wait_gather(0)
        fire_out(0, 0)
        fire_gather(RING - 1, RING - 1)

        @pl.loop(0, (n_chunks - RING) // RING)
        def _(g):
            for b in range(RING):
                i = 1 + g * RING + b
                slot = (1 + b) % RING
                prev = b
                wait_gather(slot)
                fire_out(i, slot)
                wait_out(prev)
                fire_gather(i + RING - 1, prev)

        for t in range(n_chunks - RING + 1, n_chunks):
            slot = t % RING
            wait_gather(slot)
            fire_out(t, slot)
        for b in range(RING):
            wait_out(b)

    return k


def kernel(inputs, table):
    b, s = inputs.shape
    total = b * s
    idx2d = inputs.reshape(total // CHUNK, CHUNK).astype(jnp.int32)
    out = _make_kernel(total)(table, idx2d)
    return out.reshape(b, s, DIM)


# speedup vs baseline: 4.2745x; 1.0019x over previous
import jax
import jax.numpy as jnp
from jax import lax
from jax.experimental import pallas as pl
from jax.experimental.pallas import tpu as pltpu
from jax.experimental.pallas import tpu_sc as plsc

NUM_ROWS = 100000
DIM = 64

NC = 2
NS = 16
NW = NC * NS

SPLIT = 128

RING = 8


def _make_kernel(B, S):
    nb = B // NW
    assert (nb - RING) % RING == 0

    mesh = plsc.VectorSubcoreMesh(core_axis_name="c", subcore_axis_name="s")

    @pl.kernel(
        out_type=jax.ShapeDtypeStruct((B, S, DIM), jnp.float32),
        mesh=mesh,
        scratch_types=[
            pltpu.VMEM((nb, S), jnp.int32),
            pltpu.VMEM((RING, S, DIM), jnp.float32),
        ]
        + [pltpu.SemaphoreType.DMA] * (2 * RING),
        compiler_params=pltpu.CompilerParams(use_tc_tiling_on_sc=False),
    )
    def k(table_hbm, idx_hbm, out_hbm, idx_v, bufs, *sems):
        gsem = sems[:RING]
        osem = sems[RING:]
        wid = lax.axis_index("s") * NC + lax.axis_index("c")
        b0 = wid * nb
        pltpu.sync_copy(idx_hbm.at[pl.ds(b0, nb)], idx_v)

        def fire_gather(r, slot):
            pltpu.async_copy(
                table_hbm.at[idx_v.at[r, pl.ds(0, SPLIT)]],
                bufs.at[slot, pl.ds(0, SPLIT)],
                gsem[slot],
            )
            pltpu.async_copy(
                table_hbm.at[idx_v.at[r, pl.ds(SPLIT, S - SPLIT)]],
                bufs.at[slot, pl.ds(SPLIT, S - SPLIT)],
                gsem[slot],
            )

        def wait_gather(slot):
            pltpu.make_async_copy(
                table_hbm.at[idx_v.at[0, pl.ds(0, SPLIT)]],
                bufs.at[slot, pl.ds(0, SPLIT)],
                gsem[slot],
            ).wait()
            pltpu.make_async_copy(
                table_hbm.at[idx_v.at[0, pl.ds(SPLIT, S - SPLIT)]],
                bufs.at[slot, pl.ds(SPLIT, S - SPLIT)],
                gsem[slot],
            ).wait()

        def fire_out(r, slot):
            pltpu.async_copy(bufs.at[slot], out_hbm.at[b0 + r], osem[slot])

        def wait_out(slot):
            pltpu.make_async_copy(
                bufs.at[slot], out_hbm.at[b0], osem[slot]
            ).wait()

        for t in range(RING - 1):
            fire_gather(t, t)
        wait_gather(0)
        fire_out(0, 0)
        fire_gather(RING - 1, RING - 1)

        @pl.loop(0, (nb - RING) // RING)
        def _(g):
            for b in range(RING):
                r = 1 + g * RING + b
                slot = (1 + b) % RING
                prev = b
                wait_gather(slot)
                fire_out(r, slot)
                wait_out(prev)
                fire_gather(r + RING - 1, prev)

        for t in range(nb - RING + 1, nb):
            slot = t % RING
            wait_gather(slot)
            fire_out(t, slot)
        for b in range(RING):
            wait_out(b)

    return k


def kernel(inputs, table):
    B, S = inputs.shape
    return _make_kernel(B, S)(table, inputs.astype(jnp.int32))
